# baseline (speedup 1.0000x reference)
import functools

import numpy as np
import jax
import jax.numpy as jnp
from jax import lax
from jax.experimental import pallas as pl
from jax.experimental.pallas import tpu as pltpu
from jax.experimental.pallas import tpu_sc as plsc

N = 10000
E = 160000
F = 500
D = 5000
EMB = 300
HID = 150
NL = 5
GAMMA = 1.0
EMB_RANGE = (GAMMA + 2.0) / HID

PW = 128
NP = 3
PE = NP * PW
HF = 160
PH = 640
NC, NS = 2, 16

EB = 80
BPS = 128
NBE = NS * BPS
EP = NBE * EB
ACCR = 10112
ZR = ACCR // NS
KF = 2

DB = 5120
GBAT = 80
NBD = DB // GBAT
RB = 2000
NRB = N // RB
FRB = 104
FOR = NRB * FRB

_f32 = jnp.float32
_i32 = jnp.int32


@functools.cache
def _sc_mesh():
    return plsc.VectorSubcoreMesh(
        core_axis_name="c", subcore_axis_name="s",
        num_cores=NC, num_subcores=NS)


CH = 16
NSL = 4


def _sc_scatter_body(hp0, hp1, hp2, esrc, edst, zrs,
                     agg0, agg1, agg2a, agg2b, acc,
                     sidx, didx, r0, r1, r2, r3, g0, g1, g2, g3,
                     t0, t1, t2, t3, zsem):
    c = lax.axis_index("c")
    s = lax.axis_index("s")
    rbufs = (r0, r1, r2, r3)
    gsems = (g0, g1, g2, g3)
    ssems = (t0, t1, t2, t3)

    def run(h, agg, nchunks, boff):
        zdesc = pltpu.async_copy(zrs.at[pl.ds(s * ZR, ZR)],
                                 acc.at[pl.ds(s * ZR, ZR)], zsem)

        def g_start(t, sl):
            pltpu.async_copy(h.at[sidx.at[t, 0]], rbufs[sl], gsems[sl])

        def g_wait(sl):
            pltpu.make_async_copy(
                h.at[pl.ds(0, EB)], rbufs[sl], gsems[sl]).wait()

        def s_start(t, sl):
            pltpu.async_copy(rbufs[sl], acc.at[didx.at[t, 0]], ssems[sl],
                             add=True)

        def s_wait(sl):
            pltpu.make_async_copy(
                rbufs[sl], acc.at[pl.ds(0, EB)], ssems[sl]).wait()

        for k in range(nchunks):
            cb = boff + s * (nchunks * CH) + k * CH
            pltpu.sync_copy(esrc.at[pl.ds(cb, CH)], sidx)
            pltpu.sync_copy(edst.at[pl.ds(cb, CH)], didx)
            for sl in range(NSL):
                g_start(sl, sl)
            if k == 0:
                zdesc.wait()
                plsc.subcore_barrier()

            def step(i, carry):
                base = i * NSL
                for sl in range(NSL):
                    g_wait(sl)
                    s_start(base + sl, sl)
                for sl in range(NSL):
                    s_wait(sl)

                    @pl.when(base + NSL + sl < CH)
                    def _(sl=sl, base=base):
                        g_start(base + NSL + sl, sl)

                return carry

            lax.fori_loop(0, CH // NSL, step, 0)

        plsc.subcore_barrier()
        pltpu.sync_copy(acc.at[pl.ds(s * ZR, ZR)],
                        agg.at[pl.ds(s * ZR, ZR)])

    @pl.when(c == 0)
    def _():
        run(hp0, agg0, BPS // CH, 0)
        run(hp2, agg2a, BPS // 2 // CH, 0)

    @pl.when(c == 1)
    def _():
        run(hp1, agg1, BPS // CH, 0)
        run(hp2, agg2b, BPS // 2 // CH, NBE // 2)


@functools.cache
def _sc_scatter():
    return pl.kernel(
        _sc_scatter_body,
        out_type=(jax.ShapeDtypeStruct((ACCR, PW), _f32),) * 4,
        mesh=_sc_mesh(),
        scratch_types=(
            [pltpu.VMEM_SHARED((ACCR, PW), _f32)]
            + [pltpu.VMEM((CH, 1, EB), _i32) for _ in range(2)]
            + [pltpu.VMEM((EB, PW), _f32) for _ in range(NSL)]
            + [pltpu.SemaphoreType.DMA] * (2 * NSL + 1)),
    )


NSL2 = 4


def _sc_chist_body(cdst, ccmb, ohr, zrs, c0, c1, acc, didx, cidx,
                   f0, f1, f2, f3, o0, o1, o2, o3, g0, g1, g2, g3,
                   t0, t1, t2, t3):
    c = lax.axis_index("c")
    s = lax.axis_index("s")
    bps = NBE // (NC * NS)

    fbufs = (f0, f1, f2, f3)
    obufs = (o0, o1, o2, o3)
    gsems = (g0, g1, g2, g3)
    ssems = (t0, t1, t2, t3)
    iota16 = lax.broadcasted_iota(_i32, (16,), 0)

    def run(cout, boff):
        pltpu.sync_copy(zrs.at[pl.ds(s * ZR, ZR)], acc.at[pl.ds(s * ZR, ZR)])
        plsc.subcore_barrier()

        def g_start(t, sl):
            for g in range(EB // 16):
                cb = cidx[t, 0, pl.ds(g * 16, 16)]
                fbufs[sl][0, pl.ds(g * 16, 16)] = cb * 16 + iota16 + s * 288
            pltpu.async_copy(ohr.at[fbufs[sl].at[0]], obufs[sl], gsems[sl])

        def g_wait(sl):
            pltpu.make_async_copy(
                ohr.at[pl.ds(0, EB)], obufs[sl], gsems[sl]).wait()

        def s_start(t, sl):
            pltpu.async_copy(obufs[sl], acc.at[didx.at[t, 0]], ssems[sl],
                             add=True)

        def s_wait(sl):
            pltpu.make_async_copy(
                obufs[sl], acc.at[pl.ds(0, EB)], ssems[sl]).wait()

        nchunks = bps // CH
        for k in range(nchunks):
            cb = boff + s * bps + k * CH
            pltpu.sync_copy(cdst.at[pl.ds(cb, CH)], didx)
            pltpu.sync_copy(ccmb.at[pl.ds(cb, CH)], cidx)
            for sl in range(NSL2):
                g_start(sl, sl)

            def step(i, carry):
                base = i * NSL2
                for sl in range(NSL2):
                    g_wait(sl)
                    s_start(base + sl, sl)
                for sl in range(NSL2):
                    s_wait(sl)

                    @pl.when(base + NSL2 + sl < CH)
                    def _(sl=sl, base=base):
                        g_start(base + NSL2 + sl, sl)

                return carry

            lax.fori_loop(0, CH // NSL2, step, 0)

        plsc.subcore_barrier()
        pltpu.sync_copy(acc.at[pl.ds(s * ZR, ZR)],
                        cout.at[pl.ds(s * ZR, ZR)])

    @pl.when(c == 0)
    def _():
        run(c0, 0)

    @pl.when(c == 1)
    def _():
        run(c1, NBE // 2)


@functools.cache
def _sc_chist():
    return pl.kernel(
        _sc_chist_body,
        out_type=(jax.ShapeDtypeStruct((ACCR, PW), _f32),) * 2,
        mesh=_sc_mesh(),
        scratch_types=(
            [pltpu.VMEM_SHARED((ACCR, PW), _f32)]
            + [pltpu.VMEM((CH, 1, EB), _i32) for _ in range(2)]
            + [pltpu.VMEM((1, EB), _i32) for _ in range(NSL2)]
            + [pltpu.VMEM((EB, PW), _f32) for _ in range(NSL2)]
            + [pltpu.SemaphoreType.DMA] * (2 * NSL2)),
    )


NSL3 = 3


def _sc_gather_body(ft0, ft1, ft2, hp0, hp1, hp2, du, dv, dfu, dfv,
                    f0p0, f0p1, f0p2, f1p0, f1p1, f1p2,
                    d0p0, d0p1, d0p2, d1p0, d1p1, d1p2,
                    iu, iv, ifu, ifv, r0, r1, r2,
                    g0, g1, g2, w0, w1, w2):
    c = lax.axis_index("c")
    s = lax.axis_index("s")
    w = s * NC + c
    rbufs = (r0, r1, r2)
    gsems = (g0, g1, g2)
    wsems = (w0, w1, w2)

    pltpu.sync_copy(du.at[pl.ds(w * 2, 2)], iu)
    pltpu.sync_copy(dv.at[pl.ds(w * 2, 2)], iv)
    pltpu.sync_copy(dfu.at[pl.ds(w * 2, 2)], ifu)
    pltpu.sync_copy(dfv.at[pl.ds(w * 2, 2)], ifv)

    tasks = []
    for g in range(2):
        for tab, out, ib in ((ft0, f0p0, ifu), (ft1, f0p1, ifu),
                             (ft2, f0p2, ifu), (ft0, f1p0, ifv),
                             (ft1, f1p1, ifv), (ft2, f1p2, ifv),
                             (hp0, d0p0, iu), (hp1, d0p1, iu),
                             (hp2, d0p2, iu), (hp0, d1p0, iv),
                             (hp1, d1p1, iv), (hp2, d1p2, iv)):
            tasks.append((g, tab, out, ib))
    nt = len(tasks)

    def g_start(t, sl):
        g, tab, out, ib = tasks[t]
        pltpu.async_copy(tab.at[ib.at[g, 0]], rbufs[sl], gsems[sl])

    def g_wait(sl):
        pltpu.make_async_copy(
            hp0.at[pl.ds(0, GBAT)], rbufs[sl], gsems[sl]).wait()

    def w_start(t, sl):
        g, tab, out, ib = tasks[t]
        base = (w * 2 + g) * GBAT
        pltpu.async_copy(rbufs[sl], out.at[pl.ds(base, GBAT)], wsems[sl])

    def w_wait(sl):
        pltpu.make_async_copy(
            rbufs[sl], hp0.at[pl.ds(0, GBAT)], wsems[sl]).wait()

    for sl in range(NSL3):
        g_start(sl, sl)
    for t in range(nt):
        sl = t % NSL3
        g_wait(sl)
        w_start(t, sl)
        if t + NSL3 < nt:
            w_wait(sl)
            g_start(t + NSL3, sl)
    for t in range(nt - NSL3, nt):
        w_wait(t % NSL3)


@functools.cache
def _sc_gather():
    return pl.kernel(
        _sc_gather_body,
        out_type=(jax.ShapeDtypeStruct((DB, PW), _f32),) * 12,
        mesh=_sc_mesh(),
        scratch_types=(
            [pltpu.VMEM((2, 1, GBAT), _i32) for _ in range(4)]
            + [pltpu.VMEM((GBAT, PW), _f32) for _ in range(NSL3)]
            + [pltpu.SemaphoreType.DMA] * (2 * NSL3)),
    )


def _tc_emb_body(xr, e1, e2, o0, o1, o2):
    xb = xr[...]
    oh1 = (xb[:, 0:1] == lax.broadcasted_iota(_i32, (RB, 128), 1)).astype(_f32)
    oh2 = (xb[:, 1:2] == lax.broadcasted_iota(_i32, (RB, 8), 1)).astype(_f32)
    h = (jnp.dot(oh1, e1[...], preferred_element_type=_f32)
         + jnp.dot(oh2, e2[...], preferred_element_type=_f32))
    o0[...] = h[:, :PW]
    o1[...] = h[:, PW:2 * PW]
    o2[...] = h[:, 2 * PW:]


_tc_emb = pl.pallas_call(
    _tc_emb_body,
    grid=(NRB,),
    in_specs=[pl.BlockSpec((RB, 8), lambda b: (b, 0)),
              pl.BlockSpec((128, PE), lambda b: (0, 0)),
              pl.BlockSpec((8, PE), lambda b: (0, 0))],
    out_specs=[pl.BlockSpec((RB, PW), lambda b: (b, 0))] * 3,
    out_shape=[jax.ShapeDtypeStruct((N, PW), _f32)] * 3,
)


def _make_layer_body(last):
    def body(*refs):
        if last:
            (a0, a1, a2a, a2b, h0, h1, h2p, c0, c1, ee, w1, b1, w2, b2,
             gg, bb, o0, o1, o2, fa0, fa1, fa2, h2s, sums, sqs) = refs
        else:
            (a0, a1, a2a, a2b, h0, h1, h2p, c0, c1, ee, w1, b1, w2, b2,
             gg, bb, o0, o1, o2, h2s, sums, sqs) = refs
        p = pl.program_id(0)
        b = pl.program_id(1)

        @pl.when(p == 0)
        def _compute():
            z = jnp.concatenate(
                [a0[...] + h0[...], a1[...] + h1[...],
                 a2a[...] + a2b[...] + h2p[...]], axis=1)
            z = z + jnp.dot(c0[...] + c1[...], ee[...],
                            preferred_element_type=_f32)
            hid = jnp.maximum(
                jnp.dot(z.astype(jnp.bfloat16), w1[...],
                        preferred_element_type=_f32) + b1[...], 0.0)
            h2 = jnp.dot(hid.astype(jnp.bfloat16), w2[...],
                         preferred_element_type=_f32) + b2[...]
            h2s[pl.ds(b * RB, RB), :] = h2
            colsum = jnp.sum(h2, axis=0, keepdims=True)
            colsq = jnp.sum(h2 * h2, axis=0, keepdims=True)

            @pl.when(b == 0)
            def _():
                sums[...] = colsum
                sqs[...] = colsq

            @pl.when(b > 0)
            def _():
                sums[...] += colsum
                sqs[...] += colsq

        @pl.when(p == 1)
        def _norm():
            mean = sums[...] / N
            var = sqs[...] / N - mean * mean
            rstd = lax.rsqrt(var + 1e-5)
            h2 = h2s[pl.ds(b * RB, RB), :]
            y = (h2 - mean) * rstd * gg[...] + bb[...]
            if not last:
                y = jnp.maximum(y, 0.0)
            o0[...] = y[:, :PW]
            o1[...] = y[:, PW:2 * PW]
            o2[...] = y[:, 2 * PW:]
            if last:
                rr = lax.broadcasted_iota(_i32, (FRB, RB), 0)
                cc = lax.broadcasted_iota(_i32, (FRB, RB), 1) // (N // F)
                pool = jnp.where(rr == cc, 1.0 / (N // F), 0.0).astype(_f32)
                fo = jnp.dot(pool, y, preferred_element_type=_f32)
                fa0[...] = fo[:, :PW]
                fa1[...] = fo[:, PW:2 * PW]
                fa2[...] = fo[:, 2 * PW:]

    return body


def _make_layer_call(last):
    iblk = lambda r, w: pl.BlockSpec(
        (r, w), lambda p, b: (jnp.where(p == 0, b, 0), 0))
    oblk = lambda r, w: pl.BlockSpec(
        (r, w), lambda p, b: (jnp.where(p == 1, b, 0), 0))
    full = lambda r, w: pl.BlockSpec((r, w), lambda p, b: (0, 0))
    in_specs = [iblk(RB, PW)] * 7 + [iblk(RB, PW)] * 2 + [
        full(128, PE), full(PE, PH), full(1, PH), full(PH, PE),
        full(1, PE), full(1, PE), full(1, PE)]
    out_specs = [oblk(RB, PW)] * 3
    out_shape = [jax.ShapeDtypeStruct((N, PW), _f32)] * 3
    if last:
        out_specs += [oblk(FRB, PW)] * 3
        out_shape += [jax.ShapeDtypeStruct((FOR, PW), _f32)] * 3
    return pl.pallas_call(
        _make_layer_body(last),
        grid=(2, NRB),
        in_specs=in_specs,
        out_specs=out_specs,
        out_shape=out_shape,
        scratch_shapes=[pltpu.VMEM((N, PE), _f32),
                        pltpu.VMEM((1, PE), _f32),
                        pltpu.VMEM((1, PE), _f32)],
    )


_tc_layer = _make_layer_call(False)
_tc_layer_last = _make_layer_call(True)


def _tc_proj_body(f0p0, f0p1, f0p2, d0p0, d0p1, d0p2,
                  f1p0, f1p1, f1p2, d1p0, d1p1, d1p2,
                  q0, q1, q2, q3, q4, q5, pb1, p2r, p2i, pb2r, pb2i,
                  o0r, o0i, o1r, o1i):
    qs = (q0, q1, q2, q3, q4, q5)

    def proj(parts):
        hid = pb1[...]
        for x, q in zip(parts, qs):
            hid = hid + jnp.dot(x[...].astype(jnp.bfloat16), q[...],
                                preferred_element_type=_f32)
        hid = jnp.maximum(hid, 0.0).astype(jnp.bfloat16)
        return (jnp.dot(hid, p2r[...], preferred_element_type=_f32) + pb2r[...],
                jnp.dot(hid, p2i[...], preferred_element_type=_f32) + pb2i[...])

    o0r[...], o0i[...] = proj((f0p0, f0p1, f0p2, d0p0, d0p1, d0p2))
    o1r[...], o1i[...] = proj((f1p0, f1p1, f1p2, d1p0, d1p1, d1p2))


_PRB = 1024


def _proj_call():
    blk = pl.BlockSpec((_PRB, PW), lambda b: (b, 0))
    full = lambda r, w: pl.BlockSpec((r, w), lambda b: (0, 0))
    return pl.pallas_call(
        _tc_proj_body,
        grid=(DB // _PRB,),
        in_specs=[blk] * 12 + [full(PW, PH)] * 6 + [full(1, PH)] + [
            full(PH, HF)] * 2 + [full(1, HF)] * 2,
        out_specs=[pl.BlockSpec((_PRB, HF), lambda b: (b, 0))] * 4,
        out_shape=[jax.ShapeDtypeStruct((DB, HF), _f32)] * 4,
    )


_tc_proj = _proj_call()


def _tc_score_body(o0r, o0i, t1r, t1i, t2r, t2i, da0, da1, r1, r2, pos, neg):
    oh1 = (da0[...] == lax.broadcasted_iota(_i32, (_PRB, 8), 1)).astype(_f32)
    oh2 = (da1[...] == lax.broadcasted_iota(_i32, (_PRB, 4), 1)).astype(_f32)
    rel = (jnp.dot(oh1, r1[...], preferred_element_type=_f32)
           + jnp.dot(oh2, r2[...], preferred_element_type=_f32))
    ph = rel * (np.pi / EMB_RANGE)
    cr = jnp.cos(ph)
    ci = jnp.sin(ph)
    rh = o0r[...]
    ih = o0i[...]
    for tr, ti, out in ((t1r, t1i, pos), (t2r, t2i, neg)):
        rs = rh * cr - ih * ci - tr[...]
        im = rh * ci + ih * cr - ti[...]
        sc = jnp.sqrt(rs * rs + im * im)
        out[...] = GAMMA - jnp.sum(sc, axis=1, keepdims=True)


def _score_call():
    blk = pl.BlockSpec((_PRB, HF), lambda b: (b, 0))
    iblk = lambda w: pl.BlockSpec((_PRB, w), lambda b: (b, 0))
    full = lambda r, w: pl.BlockSpec((r, w), lambda b: (0, 0))
    return pl.pallas_call(
        _tc_score_body,
        grid=(DB // _PRB,),
        in_specs=[blk] * 6 + [iblk(1), iblk(1), full(8, HF), full(4, HF)],
        out_specs=[iblk(1)] * 2,
        out_shape=[jax.ShapeDtypeStruct((DB, 1), _f32)] * 2,
    )


_tc_score = _score_call()


def _pad2(a, r, c):
    return jnp.zeros((r, c), _f32).at[:a.shape[0], :a.shape[1]].set(a)


def kernel(params, x, edge_index, edge_attr, frag_batch, frag_num_nodes,
           dangling_mask, dangling_edge_index, drop_edge_attr):
    src = edge_index[0].astype(_i32)
    dst = edge_index[1].astype(_i32)
    combo = (edge_attr[:, 0] * 3 + edge_attr[:, 1]).astype(_i32)
    padi = jnp.arange(EP - E, dtype=_i32)
    src_p = jnp.concatenate([src, padi % N])
    dst_p = jnp.concatenate([dst, N + (padi % 112)])
    combo_p = jnp.concatenate([combo, jnp.zeros(EP - E, _i32)])
    esrc = src_p.reshape(NBE, 1, EB)
    edst = dst_p.reshape(NBE, 1, EB)
    ccmb = combo_p.reshape(NBE, 1, EB)

    u = dangling_edge_index[0].astype(_i32)
    v = dangling_edge_index[1].astype(_i32)
    padd = jnp.arange(DB - D, dtype=_i32)
    up = jnp.concatenate([u, padd % N])
    vp = jnp.concatenate([v, padd % N])

    def fragrow(i):
        f = i // (N // F)
        return (f // 100) * FRB + f % 100

    rep = ((jnp.arange(DB, dtype=_i32) // (2 * GBAT)) % 4) * FOR
    du = up.reshape(NBD, 1, GBAT)
    dv = vp.reshape(NBD, 1, GBAT)
    dfu = (fragrow(up) + rep).reshape(NBD, 1, GBAT)
    dfv = (fragrow(vp) + rep).reshape(NBD, 1, GBAT)

    xpad = jnp.zeros((N, 8), _i32).at[:, :2].set(x.astype(_i32))
    da0 = jnp.zeros((DB, 1), _i32).at[:D, 0].set(drop_edge_attr[:, 0].astype(_i32))
    da1 = jnp.zeros((DB, 1), _i32).at[:D, 0].set(drop_edge_attr[:, 1].astype(_i32))

    zrs = jnp.zeros((ACCR, PW), _f32)
    ohr = jnp.tile(jnp.zeros((288, PW), _f32).at[
        jnp.arange(288), jnp.arange(288) // 16].set(1.0), (NS, 1))

    emb1p = _pad2(params['x_emb1'], 128, PE)
    emb2p = _pad2(params['x_emb2'], 8, PE)
    EE = (params['ee1'][:, :, None, :]
          + params['ee2'][:, None, :, :]).reshape(NL, 18, EMB)
    EEp = jnp.zeros((NL, 128, PE), _f32).at[:, :18, :EMB].set(EE)
    W1p = jnp.zeros((NL, PE, PH), _f32).at[
        :, :EMB, :2 * EMB].set(params['W1']).astype(jnp.bfloat16)
    b1f = params['b1'] + jnp.einsum('le,leh->lh', EE[:, 12], params['W1'])
    b1p = jnp.zeros((NL, 1, PH), _f32).at[:, 0, :2 * EMB].set(b1f)
    W2p = jnp.zeros((NL, PH, PE), _f32).at[
        :, :2 * EMB, :EMB].set(params['W2']).astype(jnp.bfloat16)
    b2p = jnp.zeros((NL, 1, PE), _f32).at[:, 0, :EMB].set(params['b2'])
    gp = jnp.zeros((NL, 1, PE), _f32).at[:, 0, :EMB].set(params['bn_g'])
    bp = jnp.zeros((NL, 1, PE), _f32).at[:, 0, :EMB].set(params['bn_b'])

    P1p = (jnp.zeros((2 * PE, PH), _f32)
           .at[:EMB, :2 * EMB].set(params['P1'][:EMB])
           .at[PE:PE + EMB, :2 * EMB].set(params['P1'][EMB:]))
    qs = tuple(P1p[k * PW:(k + 1) * PW].astype(jnp.bfloat16) for k in range(6))
    pb1p = jnp.zeros((1, PH), _f32).at[0, :2 * EMB].set(params['pb1'])
    P2r = jnp.zeros((PH, HF), _f32).at[
        :2 * EMB, :HID].set(params['P2'][:, :HID]).astype(jnp.bfloat16)
    P2i = jnp.zeros((PH, HF), _f32).at[
        :2 * EMB, :HID].set(params['P2'][:, HID:]).astype(jnp.bfloat16)
    pb2r = jnp.zeros((1, HF), _f32).at[0, :HID].set(params['pb2'][:HID])
    pb2i = jnp.zeros((1, HF), _f32).at[0, :HID].set(params['pb2'][HID:])
    rel1p = _pad2(params['rel_e1'], 8, HF)
    rel2p = _pad2(params['rel_e2'], 4, HF)

    h0, h1, h2p = _tc_emb(xpad, emb1p, emb2p)
    c0, c1 = _sc_chist()(edst, ccmb, ohr, zrs)

    for l in range(NL):
        a0, a1, a2a, a2b = _sc_scatter()(h0, h1, h2p, esrc, edst, zrs)
        args = (a0, a1, a2a, a2b, h0, h1, h2p, c0, c1, EEp[l], W1p[l],
                b1p[l], W2p[l], b2p[l], gp[l], bp[l])
        if l < NL - 1:
            h0, h1, h2p = _tc_layer(*args)
        else:
            h0, h1, h2p, ft0, ft1, ft2 = _tc_layer_last(*args)

    ft0r, ft1r, ft2r = (jnp.tile(t, (4, 1)) for t in (ft0, ft1, ft2))
    gout = _sc_gather()(ft0r, ft1r, ft2r, h0, h1, h2p, du, dv, dfu, dfv)
    (f0p0, f0p1, f0p2, f1p0, f1p1, f1p2,
     d0p0, d0p1, d0p2, d1p0, d1p1, d1p2) = gout

    o0r, o0i, o1r, o1i = _tc_proj(
        f0p0, f0p1, f0p2, d0p0, d0p1, d0p2,
        f1p0, f1p1, f1p2, d1p0, d1p1, d1p2,
        *qs, pb1p, P2r, P2i, pb2r, pb2i)

    def rollpad(o):
        return jnp.concatenate(
            [jnp.roll(o[:D], 1, axis=0), jnp.zeros((DB - D, HF), _f32)])

    o2r, o2i = rollpad(o1r), rollpad(o1i)
    pos, neg = _tc_score(o0r, o0i, o1r, o1i, o2r, o2i, da0, da1, rel1p, rel2p)
    logits = jnp.concatenate([pos[:D, 0], neg[:D, 0]])
    labels = jnp.concatenate([jnp.ones((D,), _f32), jnp.zeros((D,), _f32)])
    return logits, labels


# speedup vs baseline: 1.1067x; 1.1067x over previous
import functools

import numpy as np
import jax
import jax.numpy as jnp
from jax import lax
from jax.experimental import pallas as pl
from jax.experimental.pallas import tpu as pltpu
from jax.experimental.pallas import tpu_sc as plsc

N = 10000
E = 160000
F = 500
D = 5000
EMB = 300
HID = 150
NL = 5
GAMMA = 1.0
EMB_RANGE = (GAMMA + 2.0) / HID

PW = 128
NP = 3
PE = NP * PW
HF = 160
PH = 640
NC, NS = 2, 16

EB = 80
BPS = 128
NBE = NS * BPS
EP = NBE * EB
ACCR = 10112
ZR = ACCR // NS
KF = 2

DB = 5120
GBAT = 80
NBD = DB // GBAT
RB = 2000
NRB = N // RB
FRB = 104
FOR = NRB * FRB

_f32 = jnp.float32
_i32 = jnp.int32


@functools.cache
def _sc_mesh():
    return plsc.VectorSubcoreMesh(
        core_axis_name="c", subcore_axis_name="s",
        num_cores=NC, num_subcores=NS)


CH = 32
NSL = 4


def _sc_scatter_body(hp0, hp1, hp2, esrc, edst, zrs,
                     agg0, agg1, agg2a, agg2b, acc,
                     sidx, didx, r0, r1, r2, r3, g0, g1, g2, g3,
                     t0, t1, t2, t3, zsem, isem, jsem):
    c = lax.axis_index("c")
    s = lax.axis_index("s")
    rbufs = (r0, r1, r2, r3)
    gsems = (g0, g1, g2, g3)
    ssems = (t0, t1, t2, t3)

    def run(h, agg, nb, boff):
        zdesc = pltpu.async_copy(zrs.at[pl.ds(s * ZR, ZR)],
                                 acc.at[pl.ds(s * ZR, ZR)], zsem)
        base0 = boff + s * nb

        def g_start(t, sl):
            pltpu.async_copy(h.at[sidx.at[lax.rem(t, CH), 0]], rbufs[sl],
                             gsems[sl])

        def g_wait(sl):
            pltpu.make_async_copy(
                h.at[pl.ds(0, EB)], rbufs[sl], gsems[sl]).wait()

        def s_start(t, sl):
            pltpu.async_copy(rbufs[sl], acc.at[didx.at[lax.rem(t, CH), 0]],
                             ssems[sl], add=True)

        def s_wait(sl):
            pltpu.make_async_copy(
                rbufs[sl], acc.at[pl.ds(0, EB)], ssems[sl]).wait()

        pltpu.sync_copy(esrc.at[pl.ds(base0, CH)], sidx)
        pltpu.sync_copy(edst.at[pl.ds(base0, CH)], didx)
        for sl in range(NSL):
            g_start(sl, sl)
        zdesc.wait()
        plsc.subcore_barrier()

        def step(i, carry):
            base = i * NSL
            for sl in range(NSL):
                g_wait(sl)
                s_start(base + sl, sl)

            @pl.when(jnp.logical_and(i + 1 >= CH // NSL,
                                     base + NSL < nb))
            def _():
                pltpu.make_async_copy(
                    esrc.at[pl.ds(0, NSL)], sidx.at[pl.ds(0, NSL)],
                    isem).wait()
                pltpu.make_async_copy(
                    edst.at[pl.ds(0, NSL)], didx.at[pl.ds(0, NSL)],
                    jsem).wait()

            for sl in range(NSL):
                s_wait(sl)

                @pl.when(base + NSL + sl < nb)
                def _(sl=sl, base=base):
                    g_start(base + NSL + sl, sl)

            @pl.when(base + CH < nb)
            def _():
                sm = lax.rem(base, CH)
                pltpu.async_copy(esrc.at[pl.ds(base0 + base + CH, NSL)],
                                 sidx.at[pl.ds(sm, NSL)], isem)
                pltpu.async_copy(edst.at[pl.ds(base0 + base + CH, NSL)],
                                 didx.at[pl.ds(sm, NSL)], jsem)

            return carry

        lax.fori_loop(0, nb // NSL, step, 0)

        plsc.subcore_barrier()
        pltpu.sync_copy(acc.at[pl.ds(s * ZR, ZR)],
                        agg.at[pl.ds(s * ZR, ZR)])

    @pl.when(c == 0)
    def _():
        run(hp0, agg0, BPS, 0)
        run(hp2, agg2a, BPS // 2, 0)

    @pl.when(c == 1)
    def _():
        run(hp1, agg1, BPS, 0)
        run(hp2, agg2b, BPS // 2, NBE // 2)


@functools.cache
def _sc_scatter():
    return pl.kernel(
        _sc_scatter_body,
        out_type=(jax.ShapeDtypeStruct((ACCR, PW), _f32),) * 4,
        mesh=_sc_mesh(),
        scratch_types=(
            [pltpu.VMEM_SHARED((ACCR, PW), _f32)]
            + [pltpu.VMEM((CH, 1, EB), _i32) for _ in range(2)]
            + [pltpu.VMEM((EB, PW), _f32) for _ in range(NSL)]
            + [pltpu.SemaphoreType.DMA] * (2 * NSL + 3)),
    )


NSL2 = 4


def _sc_chist_body(cdst, ccmb, ohr, zrs, c0, c1, acc, didx, cidx,
                   f0, f1, f2, f3, o0, o1, o2, o3, g0, g1, g2, g3,
                   t0, t1, t2, t3):
    c = lax.axis_index("c")
    s = lax.axis_index("s")
    bps = NBE // (NC * NS)

    fbufs = (f0, f1, f2, f3)
    obufs = (o0, o1, o2, o3)
    gsems = (g0, g1, g2, g3)
    ssems = (t0, t1, t2, t3)
    iota16 = lax.broadcasted_iota(_i32, (16,), 0)

    def run(cout, boff):
        pltpu.sync_copy(zrs.at[pl.ds(s * ZR, ZR)], acc.at[pl.ds(s * ZR, ZR)])
        plsc.subcore_barrier()

        def g_start(t, sl):
            for g in range(EB // 16):
                cb = cidx[t, 0, pl.ds(g * 16, 16)]
                fbufs[sl][0, pl.ds(g * 16, 16)] = cb * 16 + iota16 + s * 288
            pltpu.async_copy(ohr.at[fbufs[sl].at[0]], obufs[sl], gsems[sl])

        def g_wait(sl):
            pltpu.make_async_copy(
                ohr.at[pl.ds(0, EB)], obufs[sl], gsems[sl]).wait()

        def s_start(t, sl):
            pltpu.async_copy(obufs[sl], acc.at[didx.at[t, 0]], ssems[sl],
                             add=True)

        def s_wait(sl):
            pltpu.make_async_copy(
                obufs[sl], acc.at[pl.ds(0, EB)], ssems[sl]).wait()

        nchunks = bps // CH
        for k in range(nchunks):
            cb = boff + s * bps + k * CH
            pltpu.sync_copy(cdst.at[pl.ds(cb, CH)], didx)
            pltpu.sync_copy(ccmb.at[pl.ds(cb, CH)], cidx)
            for sl in range(NSL2):
                g_start(sl, sl)

            def step(i, carry):
                base = i * NSL2
                for sl in range(NSL2):
                    g_wait(sl)
                    s_start(base + sl, sl)
                for sl in range(NSL2):
                    s_wait(sl)

                    @pl.when(base + NSL2 + sl < CH)
                    def _(sl=sl, base=base):
                        g_start(base + NSL2 + sl, sl)

                return carry

            lax.fori_loop(0, CH // NSL2, step, 0)

        plsc.subcore_barrier()
        pltpu.sync_copy(acc.at[pl.ds(s * ZR, ZR)],
                        cout.at[pl.ds(s * ZR, ZR)])

    @pl.when(c == 0)
    def _():
        run(c0, 0)

    @pl.when(c == 1)
    def _():
        run(c1, NBE // 2)


@functools.cache
def _sc_chist():
    return pl.kernel(
        _sc_chist_body,
        out_type=(jax.ShapeDtypeStruct((ACCR, PW), _f32),) * 2,
        mesh=_sc_mesh(),
        scratch_types=(
            [pltpu.VMEM_SHARED((ACCR, PW), _f32)]
            + [pltpu.VMEM((CH, 1, EB), _i32) for _ in range(2)]
            + [pltpu.VMEM((1, EB), _i32) for _ in range(NSL2)]
            + [pltpu.VMEM((EB, PW), _f32) for _ in range(NSL2)]
            + [pltpu.SemaphoreType.DMA] * (2 * NSL2)),
    )


NSL3 = 3


def _sc_gather_body(ft0, ft1, ft2, hp0, hp1, hp2, du, dv, dfu, dfv,
                    f0p0, f0p1, f0p2, f1p0, f1p1, f1p2,
                    d0p0, d0p1, d0p2, d1p0, d1p1, d1p2,
                    iu, iv, ifu, ifv, r0, r1, r2,
                    g0, g1, g2, w0, w1, w2):
    c = lax.axis_index("c")
    s = lax.axis_index("s")
    w = s * NC + c
    rbufs = (r0, r1, r2)
    gsems = (g0, g1, g2)
    wsems = (w0, w1, w2)

    pltpu.sync_copy(du.at[pl.ds(w * 2, 2)], iu)
    pltpu.sync_copy(dv.at[pl.ds(w * 2, 2)], iv)
    pltpu.sync_copy(dfu.at[pl.ds(w * 2, 2)], ifu)
    pltpu.sync_copy(dfv.at[pl.ds(w * 2, 2)], ifv)

    tasks = []
    for g in range(2):
        for tab, out, ib in ((ft0, f0p0, ifu), (ft1, f0p1, ifu),
                             (ft2, f0p2, ifu), (ft0, f1p0, ifv),
                             (ft1, f1p1, ifv), (ft2, f1p2, ifv),
                             (hp0, d0p0, iu), (hp1, d0p1, iu),
                             (hp2, d0p2, iu), (hp0, d1p0, iv),
                             (hp1, d1p1, iv), (hp2, d1p2, iv)):
            tasks.append((g, tab, out, ib))
    nt = len(tasks)

    def g_start(t, sl):
        g, tab, out, ib = tasks[t]
        pltpu.async_copy(tab.at[ib.at[g, 0]], rbufs[sl], gsems[sl])

    def g_wait(sl):
        pltpu.make_async_copy(
            hp0.at[pl.ds(0, GBAT)], rbufs[sl], gsems[sl]).wait()

    def w_start(t, sl):
        g, tab, out, ib = tasks[t]
        base = (w * 2 + g) * GBAT
        pltpu.async_copy(rbufs[sl], out.at[pl.ds(base, GBAT)], wsems[sl])

    def w_wait(sl):
        pltpu.make_async_copy(
            rbufs[sl], hp0.at[pl.ds(0, GBAT)], wsems[sl]).wait()

    for sl in range(NSL3):
        g_start(sl, sl)
    for t in range(nt):
        sl = t % NSL3
        g_wait(sl)
        w_start(t, sl)
        if t + NSL3 < nt:
            w_wait(sl)
            g_start(t + NSL3, sl)
    for t in range(nt - NSL3, nt):
        w_wait(t % NSL3)


@functools.cache
def _sc_gather():
    return pl.kernel(
        _sc_gather_body,
        out_type=(jax.ShapeDtypeStruct((DB, PW), _f32),) * 12,
        mesh=_sc_mesh(),
        scratch_types=(
            [pltpu.VMEM((2, 1, GBAT), _i32) for _ in range(4)]
            + [pltpu.VMEM((GBAT, PW), _f32) for _ in range(NSL3)]
            + [pltpu.SemaphoreType.DMA] * (2 * NSL3)),
    )


def _tc_emb_body(xr, e1, e2, o0, o1, o2):
    xb = xr[...]
    oh1 = (xb[:, 0:1] == lax.broadcasted_iota(_i32, (RB, 128), 1)).astype(_f32)
    oh2 = (xb[:, 1:2] == lax.broadcasted_iota(_i32, (RB, 8), 1)).astype(_f32)
    h = (jnp.dot(oh1, e1[...], preferred_element_type=_f32)
         + jnp.dot(oh2, e2[...], preferred_element_type=_f32))
    o0[...] = h[:, :PW]
    o1[...] = h[:, PW:2 * PW]
    o2[...] = h[:, 2 * PW:]


_tc_emb = pl.pallas_call(
    _tc_emb_body,
    grid=(NRB,),
    in_specs=[pl.BlockSpec((RB, 8), lambda b: (b, 0)),
              pl.BlockSpec((128, PE), lambda b: (0, 0)),
              pl.BlockSpec((8, PE), lambda b: (0, 0))],
    out_specs=[pl.BlockSpec((RB, PW), lambda b: (b, 0))] * 3,
    out_shape=[jax.ShapeDtypeStruct((N, PW), _f32)] * 3,
)


def _make_layer_body(last):
    def body(*refs):
        if last:
            (a0, a1, a2a, a2b, h0, h1, h2p, c0, c1, ee, w1, b1, w2, b2,
             gg, bb, o0, o1, o2, fa0, fa1, fa2, h2s, sums, sqs) = refs
        else:
            (a0, a1, a2a, a2b, h0, h1, h2p, c0, c1, ee, w1, b1, w2, b2,
             gg, bb, o0, o1, o2, h2s, sums, sqs) = refs
        p = pl.program_id(0)
        b = pl.program_id(1)

        @pl.when(p == 0)
        def _compute():
            z = jnp.concatenate(
                [a0[...] + h0[...], a1[...] + h1[...],
                 a2a[...] + a2b[...] + h2p[...]], axis=1)
            z = z + jnp.dot(c0[...] + c1[...], ee[...],
                            preferred_element_type=_f32)
            hid = jnp.maximum(
                jnp.dot(z.astype(jnp.bfloat16), w1[...],
                        preferred_element_type=_f32) + b1[...], 0.0)
            h2 = jnp.dot(hid.astype(jnp.bfloat16), w2[...],
                         preferred_element_type=_f32) + b2[...]
            h2s[pl.ds(b * RB, RB), :] = h2
            colsum = jnp.sum(h2, axis=0, keepdims=True)
            colsq = jnp.sum(h2 * h2, axis=0, keepdims=True)

            @pl.when(b == 0)
            def _():
                sums[...] = colsum
                sqs[...] = colsq

            @pl.when(b > 0)
            def _():
                sums[...] += colsum
                sqs[...] += colsq

        @pl.when(p == 1)
        def _norm():
            mean = sums[...] / N
            var = sqs[...] / N - mean * mean
            rstd = lax.rsqrt(var + 1e-5)
            h2 = h2s[pl.ds(b * RB, RB), :]
            y = (h2 - mean) * rstd * gg[...] + bb[...]
            if not last:
                y = jnp.maximum(y, 0.0)
            o0[...] = y[:, :PW]
            o1[...] = y[:, PW:2 * PW]
            o2[...] = y[:, 2 * PW:]
            if last:
                rr = lax.broadcasted_iota(_i32, (FRB, RB), 0)
                cc = lax.broadcasted_iota(_i32, (FRB, RB), 1) // (N // F)
                pool = jnp.where(rr == cc, 1.0 / (N // F), 0.0).astype(_f32)
                fo = jnp.dot(pool, y, preferred_element_type=_f32)
                fa0[...] = fo[:, :PW]
                fa1[...] = fo[:, PW:2 * PW]
                fa2[...] = fo[:, 2 * PW:]

    return body


def _make_layer_call(last):
    iblk = lambda r, w: pl.BlockSpec(
        (r, w), lambda p, b: (jnp.where(p == 0, b, 0), 0))
    oblk = lambda r, w: pl.BlockSpec(
        (r, w), lambda p, b: (jnp.where(p == 1, b, 0), 0))
    full = lambda r, w: pl.BlockSpec((r, w), lambda p, b: (0, 0))
    in_specs = [iblk(RB, PW)] * 7 + [iblk(RB, PW)] * 2 + [
        full(128, PE), full(PE, PH), full(1, PH), full(PH, PE),
        full(1, PE), full(1, PE), full(1, PE)]
    out_specs = [oblk(RB, PW)] * 3
    out_shape = [jax.ShapeDtypeStruct((N, PW), _f32)] * 3
    if last:
        out_specs += [oblk(FRB, PW)] * 3
        out_shape += [jax.ShapeDtypeStruct((FOR, PW), _f32)] * 3
    return pl.pallas_call(
        _make_layer_body(last),
        grid=(2, NRB),
        in_specs=in_specs,
        out_specs=out_specs,
        out_shape=out_shape,
        scratch_shapes=[pltpu.VMEM((N, PE), _f32),
                        pltpu.VMEM((1, PE), _f32),
                        pltpu.VMEM((1, PE), _f32)],
    )


_tc_layer = _make_layer_call(False)
_tc_layer_last = _make_layer_call(True)


def _tc_proj_body(f0p0, f0p1, f0p2, d0p0, d0p1, d0p2,
                  f1p0, f1p1, f1p2, d1p0, d1p1, d1p2,
                  q0, q1, q2, q3, q4, q5, pb1, p2r, p2i, pb2r, pb2i,
                  o0r, o0i, o1r, o1i):
    qs = (q0, q1, q2, q3, q4, q5)

    def proj(parts):
        hid = pb1[...]
        for x, q in zip(parts, qs):
            hid = hid + jnp.dot(x[...].astype(jnp.bfloat16), q[...],
                                preferred_element_type=_f32)
        hid = jnp.maximum(hid, 0.0).astype(jnp.bfloat16)
        return (jnp.dot(hid, p2r[...], preferred_element_type=_f32) + pb2r[...],
                jnp.dot(hid, p2i[...], preferred_element_type=_f32) + pb2i[...])

    o0r[...], o0i[...] = proj((f0p0, f0p1, f0p2, d0p0, d0p1, d0p2))
    o1r[...], o1i[...] = proj((f1p0, f1p1, f1p2, d1p0, d1p1, d1p2))


_PRB = 1024


def _proj_call():
    blk = pl.BlockSpec((_PRB, PW), lambda b: (b, 0))
    full = lambda r, w: pl.BlockSpec((r, w), lambda b: (0, 0))
    return pl.pallas_call(
        _tc_proj_body,
        grid=(DB // _PRB,),
        in_specs=[blk] * 12 + [full(PW, PH)] * 6 + [full(1, PH)] + [
            full(PH, HF)] * 2 + [full(1, HF)] * 2,
        out_specs=[pl.BlockSpec((_PRB, HF), lambda b: (b, 0))] * 4,
        out_shape=[jax.ShapeDtypeStruct((DB, HF), _f32)] * 4,
    )


_tc_proj = _proj_call()


def _tc_score_body(o0r, o0i, t1r, t1i, t2r, t2i, da0, da1, r1, r2, pos, neg):
    oh1 = (da0[...] == lax.broadcasted_iota(_i32, (_PRB, 8), 1)).astype(_f32)
    oh2 = (da1[...] == lax.broadcasted_iota(_i32, (_PRB, 4), 1)).astype(_f32)
    rel = (jnp.dot(oh1, r1[...], preferred_element_type=_f32)
           + jnp.dot(oh2, r2[...], preferred_element_type=_f32))
    ph = rel * (np.pi / EMB_RANGE)
    cr = jnp.cos(ph)
    ci = jnp.sin(ph)
    rh = o0r[...]
    ih = o0i[...]
    for tr, ti, out in ((t1r, t1i, pos), (t2r, t2i, neg)):
        rs = rh * cr - ih * ci - tr[...]
        im = rh * ci + ih * cr - ti[...]
        sc = jnp.sqrt(rs * rs + im * im)
        out[...] = GAMMA - jnp.sum(sc, axis=1, keepdims=True)


def _score_call():
    blk = pl.BlockSpec((_PRB, HF), lambda b: (b, 0))
    iblk = lambda w: pl.BlockSpec((_PRB, w), lambda b: (b, 0))
    full = lambda r, w: pl.BlockSpec((r, w), lambda b: (0, 0))
    return pl.pallas_call(
        _tc_score_body,
        grid=(DB // _PRB,),
        in_specs=[blk] * 6 + [iblk(1), iblk(1), full(8, HF), full(4, HF)],
        out_specs=[iblk(1)] * 2,
        out_shape=[jax.ShapeDtypeStruct((DB, 1), _f32)] * 2,
    )


_tc_score = _score_call()


def _pad2(a, r, c):
    return jnp.zeros((r, c), _f32).at[:a.shape[0], :a.shape[1]].set(a)


def kernel(params, x, edge_index, edge_attr, frag_batch, frag_num_nodes,
           dangling_mask, dangling_edge_index, drop_edge_attr):
    src = edge_index[0].astype(_i32)
    dst = edge_index[1].astype(_i32)
    combo = (edge_attr[:, 0] * 3 + edge_attr[:, 1]).astype(_i32)
    padi = jnp.arange(EP - E, dtype=_i32)
    src_p = jnp.concatenate([src, padi % N])
    dst_p = jnp.concatenate([dst, N + (padi % 112)])
    combo_p = jnp.concatenate([combo, jnp.zeros(EP - E, _i32)])
    esrc = src_p.reshape(NBE, 1, EB)
    edst = dst_p.reshape(NBE, 1, EB)
    ccmb = combo_p.reshape(NBE, 1, EB)

    u = dangling_edge_index[0].astype(_i32)
    v = dangling_edge_index[1].astype(_i32)
    padd = jnp.arange(DB - D, dtype=_i32)
    up = jnp.concatenate([u, padd % N])
    vp = jnp.concatenate([v, padd % N])

    def fragrow(i):
        f = i // (N // F)
        return (f // 100) * FRB + f % 100

    rep = ((jnp.arange(DB, dtype=_i32) // (2 * GBAT)) % 4) * FOR
    du = up.reshape(NBD, 1, GBAT)
    dv = vp.reshape(NBD, 1, GBAT)
    dfu = (fragrow(up) + rep).reshape(NBD, 1, GBAT)
    dfv = (fragrow(vp) + rep).reshape(NBD, 1, GBAT)

    xpad = jnp.zeros((N, 8), _i32).at[:, :2].set(x.astype(_i32))
    da0 = jnp.zeros((DB, 1), _i32).at[:D, 0].set(drop_edge_attr[:, 0].astype(_i32))
    da1 = jnp.zeros((DB, 1), _i32).at[:D, 0].set(drop_edge_attr[:, 1].astype(_i32))

    zrs = jnp.zeros((ACCR, PW), _f32)
    ohr = jnp.tile(jnp.zeros((288, PW), _f32).at[
        jnp.arange(288), jnp.arange(288) // 16].set(1.0), (NS, 1))

    emb1p = _pad2(params['x_emb1'], 128, PE)
    emb2p = _pad2(params['x_emb2'], 8, PE)
    EE = (params['ee1'][:, :, None, :]
          + params['ee2'][:, None, :, :]).reshape(NL, 18, EMB)
    EEp = jnp.zeros((NL, 128, PE), _f32).at[:, :18, :EMB].set(EE)
    W1p = jnp.zeros((NL, PE, PH), _f32).at[
        :, :EMB, :2 * EMB].set(params['W1']).astype(jnp.bfloat16)
    b1f = params['b1'] + jnp.einsum('le,leh->lh', EE[:, 12], params['W1'])
    b1p = jnp.zeros((NL, 1, PH), _f32).at[:, 0, :2 * EMB].set(b1f)
    W2p = jnp.zeros((NL, PH, PE), _f32).at[
        :, :2 * EMB, :EMB].set(params['W2']).astype(jnp.bfloat16)
    b2p = jnp.zeros((NL, 1, PE), _f32).at[:, 0, :EMB].set(params['b2'])
    gp = jnp.zeros((NL, 1, PE), _f32).at[:, 0, :EMB].set(params['bn_g'])
    bp = jnp.zeros((NL, 1, PE), _f32).at[:, 0, :EMB].set(params['bn_b'])

    P1p = (jnp.zeros((2 * PE, PH), _f32)
           .at[:EMB, :2 * EMB].set(params['P1'][:EMB])
           .at[PE:PE + EMB, :2 * EMB].set(params['P1'][EMB:]))
    qs = tuple(P1p[k * PW:(k + 1) * PW].astype(jnp.bfloat16) for k in range(6))
    pb1p = jnp.zeros((1, PH), _f32).at[0, :2 * EMB].set(params['pb1'])
    P2r = jnp.zeros((PH, HF), _f32).at[
        :2 * EMB, :HID].set(params['P2'][:, :HID]).astype(jnp.bfloat16)
    P2i = jnp.zeros((PH, HF), _f32).at[
        :2 * EMB, :HID].set(params['P2'][:, HID:]).astype(jnp.bfloat16)
    pb2r = jnp.zeros((1, HF), _f32).at[0, :HID].set(params['pb2'][:HID])
    pb2i = jnp.zeros((1, HF), _f32).at[0, :HID].set(params['pb2'][HID:])
    rel1p = _pad2(params['rel_e1'], 8, HF)
    rel2p = _pad2(params['rel_e2'], 4, HF)

    h0, h1, h2p = _tc_emb(xpad, emb1p, emb2p)
    c0, c1 = _sc_chist()(edst, ccmb, ohr, zrs)

    for l in range(NL):
        a0, a1, a2a, a2b = _sc_scatter()(h0, h1, h2p, esrc, edst, zrs)
        args = (a0, a1, a2a, a2b, h0, h1, h2p, c0, c1, EEp[l], W1p[l],
                b1p[l], W2p[l], b2p[l], gp[l], bp[l])
        if l < NL - 1:
            h0, h1, h2p = _tc_layer(*args)
        else:
            h0, h1, h2p, ft0, ft1, ft2 = _tc_layer_last(*args)

    ft0r, ft1r, ft2r = (jnp.tile(t, (4, 1)) for t in (ft0, ft1, ft2))
    gout = _sc_gather()(ft0r, ft1r, ft2r, h0, h1, h2p, du, dv, dfu, dfv)
    (f0p0, f0p1, f0p2, f1p0, f1p1, f1p2,
     d0p0, d0p1, d0p2, d1p0, d1p1, d1p2) = gout

    o0r, o0i, o1r, o1i = _tc_proj(
        f0p0, f0p1, f0p2, d0p0, d0p1, d0p2,
        f1p0, f1p1, f1p2, d1p0, d1p1, d1p2,
        *qs, pb1p, P2r, P2i, pb2r, pb2i)

    def rollpad(o):
        return jnp.concatenate(
            [jnp.roll(o[:D], 1, axis=0), jnp.zeros((DB - D, HF), _f32)])

    o2r, o2i = rollpad(o1r), rollpad(o1i)
    pos, neg = _tc_score(o0r, o0i, o1r, o1i, o2r, o2i, da0, da1, rel1p, rel2p)
    logits = jnp.concatenate([pos[:D, 0], neg[:D, 0]])
    labels = jnp.concatenate([jnp.ones((D,), _f32), jnp.zeros((D,), _f32)])
    return logits, labels
